# baseline (speedup 1.0000x reference)
import functools

import jax
import jax.numpy as jnp
from jax import lax
from jax.experimental import pallas as pl
from jax.experimental.pallas import tpu as pltpu
from jax.experimental.pallas import tpu_sc as plsc

N_NODE = 10000
E_EDGES = 160000
H = 256
HH = 128
D_E = 16
N_LAYERS_K = 2
NTILE = 16
EPT = E_EDGES // NTILE

SCH = 80
SNCH = EPT // SCH

PCH = 80
PNCH = EPT // PCH

ALN = 624
TAIL = N_NODE - ALN * NTILE

_MESH = plsc.VectorSubcoreMesh(core_axis_name="c", subcore_axis_name="s")


def _tiled_init(s, zeros, accs):
    for acc in accs:
        pltpu.sync_copy(zeros.at[pl.ds(0, ALN)], acc.at[pl.ds(s * ALN, ALN)])

    @pl.when(s == NTILE - 1)
    def _():
        for acc in accs:
            pltpu.sync_copy(zeros.at[pl.ds(0, TAIL)],
                            acc.at[pl.ds(ALN * NTILE, TAIL)])


def _tiled_copy_out(s, outs):
    for acc, out_slicer in outs:
        pltpu.sync_copy(acc.at[pl.ds(s * ALN, ALN)], out_slicer(s * ALN, ALN))

    @pl.when(s == NTILE - 1)
    def _():
        for acc, out_slicer in outs:
            pltpu.sync_copy(acc.at[pl.ds(ALN * NTILE, TAIL)],
                            out_slicer(ALN * NTILE, TAIL))


def _segsum2_body(htab_cp, srcidx_cp, dstidx_cp, htab_pc, srcidx_pc,
                  dstidx_pc, zeros, out_cp, out_pc,
                  src_v, dst_v, buf, accum, gsem):
    c = lax.axis_index("c")
    s = lax.axis_index("s")

    def one_etype(htable, srcidx, dstidx, out):
        pltpu.sync_copy(srcidx.at[c, s], src_v)
        pltpu.sync_copy(dstidx.at[s], dst_v)
        _tiled_init(s, zeros, [accum])
        plsc.subcore_barrier()

        def body(g, carry):
            pltpu.async_copy(htable.at[src_v.at[g]], buf, gsem).wait()
            pltpu.sync_copy(buf, accum.at[dst_v.at[g]], add=True)
            return carry

        lax.fori_loop(0, SNCH, body, 0)
        plsc.subcore_barrier()
        _tiled_copy_out(s, [(accum, lambda o, n: out.at[c, pl.ds(o, n)])])

    one_etype(htab_cp, srcidx_cp, dstidx_cp, out_cp)
    plsc.subcore_barrier()
    one_etype(htab_pc, srcidx_pc, dstidx_pc, out_pc)


_segsum2 = functools.partial(
    pl.kernel,
    out_type=(jax.ShapeDtypeStruct((2, N_NODE, HH), jnp.float32),
              jax.ShapeDtypeStruct((2, N_NODE, HH), jnp.float32)),
    mesh=_MESH,
    scratch_types=[
        pltpu.VMEM((SNCH, SCH), jnp.int32),
        pltpu.VMEM((SNCH, SCH), jnp.int32),
        pltpu.VMEM((SCH, HH), jnp.float32),
        pltpu.VMEM_SHARED((N_NODE, HH), jnp.float32),
        pltpu.SemaphoreType.DMA,
    ],
)(_segsum2_body)


def _pre_body(efeat, dstidx, zeros, zeros_chunk, sd_out,
              dst_v, fbuf, staging, accum, gsem):
    c = lax.axis_index("c")
    s = lax.axis_index("s")
    pltpu.sync_copy(dstidx.at[c, s], dst_v)
    pltpu.sync_copy(zeros_chunk, staging)
    one_hot = jnp.where(lax.iota(jnp.int32, 16) == 0,
                        jnp.float32(1.0), jnp.float32(0.0))

    def set_ones(j, carry):
        staging[j, pl.ds(D_E, 16)] = one_hot
        return carry

    lax.fori_loop(0, PCH, set_ones, 0)
    _tiled_init(s, zeros, [accum])
    plsc.subcore_barrier()

    def body(g, carry):
        pltpu.async_copy(efeat.at[c, s, g], fbuf, gsem).wait()

        def place(j, carry2):
            staging[j, pl.ds(0, D_E)] = fbuf[pl.ds(j * D_E, D_E)]
            return carry2

        lax.fori_loop(0, PCH, place, 0)
        pltpu.sync_copy(staging, accum.at[dst_v.at[g]], add=True)
        return carry

    lax.fori_loop(0, PNCH, body, 0)
    plsc.subcore_barrier()
    _tiled_copy_out(s, [(accum, lambda o, n: sd_out.at[c, pl.ds(o, n)])])


_precompute = functools.partial(
    pl.kernel,
    out_type=jax.ShapeDtypeStruct((2, N_NODE, HH), jnp.float32),
    mesh=_MESH,
    scratch_types=[
        pltpu.VMEM((PNCH, PCH), jnp.int32),
        pltpu.VMEM((PCH * D_E,), jnp.float32),
        pltpu.VMEM((PCH, HH), jnp.float32),
        pltpu.VMEM_SHARED((N_NODE, HH), jnp.float32),
        pltpu.SemaphoreType.DMA,
    ],
)(_pre_body)


_BN = 2000


def _embed_body(h_ref, w_ref, b_ref, out_ref):
    x = jnp.dot(h_ref[...], w_ref[...],
                preferred_element_type=jnp.float32) + b_ref[...]
    out_ref[0] = x[:, :HH]
    out_ref[1] = x[:, HH:]


def _embed(h, W, b):
    n, d = h.shape
    return pl.pallas_call(
        _embed_body,
        grid=(n // _BN,),
        in_specs=[pl.BlockSpec((_BN, d), lambda i: (i, 0)),
                  pl.BlockSpec((d, H), lambda i: (0, 0)),
                  pl.BlockSpec((1, H), lambda i: (0, 0))],
        out_specs=pl.BlockSpec((2, _BN, HH), lambda i: (0, i, 0)),
        out_shape=jax.ShapeDtypeStruct((2, n, HH), jnp.float32),
    )(h, W, b.reshape(1, H))


def _conv_update_body(split_out, h2, s2, sd, wm, ws, wn, bm, bs, bn,
                      out_ref):
    Wm = wm[...]
    h = jnp.concatenate([h2[0], h2[1]], axis=1)
    Sg = jnp.concatenate([s2[0], s2[1]], axis=1)
    sdv = sd[...]
    sef = sdv[:, :D_E]
    degv = sdv[:, D_E:D_E + 1]
    agg_u = (jnp.dot(Sg, Wm[:H], preferred_element_type=jnp.float32)
             + jnp.dot(sef, Wm[H:], preferred_element_type=jnp.float32)
             + degv * bm[...])
    agg = agg_u / jnp.maximum(degv, 1.0)
    x = (jnp.dot(h, ws[...], preferred_element_type=jnp.float32)
         + jnp.dot(agg, wn[...], preferred_element_type=jnp.float32)
         + bs[...] + bn[...])
    x = jnp.maximum(x, 0.0)
    nrm = jnp.sqrt(jnp.sum(x * x, axis=1, keepdims=True))
    x = x / (nrm + 1e-6)
    if split_out:
        out_ref[0] = x[:, :HH]
        out_ref[1] = x[:, HH:]
    else:
        out_ref[...] = x


def _conv_update(h2, s2, sd, Wm, bm, Ws, bs, Wn, bn, split_out):
    if split_out:
        out_spec = pl.BlockSpec((2, _BN, HH), lambda i: (0, i, 0))
        out_shape = jax.ShapeDtypeStruct((2, N_NODE, HH), jnp.float32)
    else:
        out_spec = pl.BlockSpec((_BN, H), lambda i: (i, 0))
        out_shape = jax.ShapeDtypeStruct((N_NODE, H), jnp.float32)
    return pl.pallas_call(
        functools.partial(_conv_update_body, split_out),
        grid=(N_NODE // _BN,),
        in_specs=[pl.BlockSpec((2, _BN, HH), lambda i: (0, i, 0)),
                  pl.BlockSpec((2, _BN, HH), lambda i: (0, i, 0)),
                  pl.BlockSpec((_BN, HH), lambda i: (i, 0)),
                  pl.BlockSpec((H + D_E, H), lambda i: (0, 0)),
                  pl.BlockSpec((H, H), lambda i: (0, 0)),
                  pl.BlockSpec((H, H), lambda i: (0, 0)),
                  pl.BlockSpec((1, H), lambda i: (0, 0)),
                  pl.BlockSpec((1, H), lambda i: (0, 0)),
                  pl.BlockSpec((1, H), lambda i: (0, 0))],
        out_specs=out_spec,
        out_shape=out_shape,
    )(h2, s2, sd, Wm, Ws, Wn,
      bm.reshape(1, H), bs.reshape(1, H), bn.reshape(1, H))


def kernel(h_customer, h_product, edge_index_c2p, edge_index_p2c,
           edge_feat_c2p, edge_feat_p2c,
           W_user, b_user, W_item, b_item,
           W_msg_c2p, b_msg_c2p, W_self_c2p, b_self_c2p, W_neigh_c2p, b_neigh_c2p,
           W_msg_p2c, b_msg_p2c, W_self_p2c, b_self_p2c, W_neigh_p2c, b_neigh_p2c):
    f32 = jnp.float32
    i32 = jnp.int32
    N = N_NODE

    src_cp = edge_index_c2p[0].astype(i32)
    dst_cp = edge_index_c2p[1].astype(i32)
    src_pc = edge_index_p2c[0].astype(i32)
    dst_pc = edge_index_p2c[1].astype(i32)

    srcidx_cp = jnp.stack([src_cp, src_cp + N]).reshape(2, NTILE, SNCH, SCH)
    srcidx_pc = jnp.stack([src_pc, src_pc + N]).reshape(2, NTILE, SNCH, SCH)
    dstidx_cp = dst_cp.reshape(NTILE, SNCH, SCH)
    dstidx_pc = dst_pc.reshape(NTILE, SNCH, SCH)

    efeat_both = jnp.stack([edge_feat_c2p.reshape(NTILE, PNCH, PCH * D_E),
                            edge_feat_p2c.reshape(NTILE, PNCH, PCH * D_E)])
    pre_dst_both = jnp.stack([dst_cp.reshape(NTILE, PNCH, PCH),
                              dst_pc.reshape(NTILE, PNCH, PCH)])
    zeros_hh = jnp.zeros((ALN, HH), f32)

    sefdeg = _precompute(efeat_both, pre_dst_both, zeros_hh,
                         jnp.zeros((PCH, HH), f32))
    sd_p = sefdeg[0]
    sd_c = sefdeg[1]

    hc2 = _embed(h_customer, W_user, b_user)
    hp2 = _embed(h_product, W_item, b_item)

    prev_sc = sefdeg
    for l in range(N_LAYERS_K):
        split = l < N_LAYERS_K - 1
        hc2_d, _ = lax.optimization_barrier((hc2, prev_sc))
        s_cp, s_pc = _segsum2(hc2_d.reshape(2 * N, HH), srcidx_cp,
                              dstidx_cp, hp2.reshape(2 * N, HH),
                              srcidx_pc, dstidx_pc, zeros_hh)
        prev_sc = s_pc
        new_hp = _conv_update(hp2, s_cp, sd_p,
                              W_msg_c2p[l], b_msg_c2p[l], W_self_c2p[l],
                              b_self_c2p[l], W_neigh_c2p[l], b_neigh_c2p[l],
                              split)
        new_hc = _conv_update(hc2, s_pc, sd_c,
                              W_msg_p2c[l], b_msg_p2c[l], W_self_p2c[l],
                              b_self_p2c[l], W_neigh_p2c[l], b_neigh_p2c[l],
                              split)
        hc2, hp2 = new_hc, new_hp

    return hc2, hp2


# speedup vs baseline: 1.1764x; 1.1764x over previous
import functools

import jax
import jax.numpy as jnp
from jax import lax
from jax.experimental import pallas as pl
from jax.experimental.pallas import tpu as pltpu
from jax.experimental.pallas import tpu_sc as plsc

N_NODE = 10000
E_EDGES = 160000
H = 256
HH = 128
D_E = 16
N_LAYERS_K = 2
NTILE = 16
EPT = E_EDGES // NTILE

SCH = 80
SNCH = EPT // SCH

PCH = 80
PNCH = EPT // PCH

ALN = 624
TAIL = N_NODE - ALN * NTILE

_MESH = plsc.VectorSubcoreMesh(core_axis_name="c", subcore_axis_name="s")


def _tiled_init(s, zeros, accs):
    for acc in accs:
        pltpu.sync_copy(zeros.at[pl.ds(0, ALN)], acc.at[pl.ds(s * ALN, ALN)])

    @pl.when(s == NTILE - 1)
    def _():
        for acc in accs:
            pltpu.sync_copy(zeros.at[pl.ds(0, TAIL)],
                            acc.at[pl.ds(ALN * NTILE, TAIL)])


def _tiled_copy_out(s, outs):
    for acc, out_slicer in outs:
        pltpu.sync_copy(acc.at[pl.ds(s * ALN, ALN)], out_slicer(s * ALN, ALN))

    @pl.when(s == NTILE - 1)
    def _():
        for acc, out_slicer in outs:
            pltpu.sync_copy(acc.at[pl.ds(ALN * NTILE, TAIL)],
                            out_slicer(ALN * NTILE, TAIL))


def _segsum_body(htable, srcidx, dstidx, zeros, out,
                 src_v, dst_v, buf_a, buf_b, accum, gsem_a, gsem_b):
    c = lax.axis_index("c")
    s = lax.axis_index("s")
    pltpu.sync_copy(srcidx.at[c, s], src_v)
    pltpu.sync_copy(dstidx.at[s], dst_v)
    _tiled_init(s, zeros, [accum])
    plsc.subcore_barrier()

    def body(t, carry):
        g = t * 2
        d0 = pltpu.async_copy(htable.at[src_v.at[pl.ds(g * SCH, SCH)]],
                              buf_a, gsem_a)
        d1 = pltpu.async_copy(htable.at[src_v.at[pl.ds((g + 1) * SCH, SCH)]],
                              buf_b, gsem_b)
        d0.wait()
        pltpu.sync_copy(buf_a, accum.at[dst_v.at[g]], add=True)
        d1.wait()
        pltpu.sync_copy(buf_b, accum.at[dst_v.at[g + 1]], add=True)
        return carry

    lax.fori_loop(0, SNCH // 2, body, 0)

    dl = pltpu.async_copy(htable.at[src_v.at[pl.ds((SNCH - 1) * SCH, SCH)]],
                          buf_a, gsem_a)
    dl.wait()
    pltpu.sync_copy(buf_a, accum.at[dst_v.at[SNCH - 1]], add=True)

    plsc.subcore_barrier()
    _tiled_copy_out(s, [(accum, lambda o, n: out.at[c, pl.ds(o, n)])])


_segsum = functools.partial(
    pl.kernel,
    out_type=jax.ShapeDtypeStruct((2, N_NODE, HH), jnp.float32),
    mesh=_MESH,
    scratch_types=[
        pltpu.VMEM((EPT,), jnp.int32),
        pltpu.VMEM((SNCH, SCH), jnp.int32),
        pltpu.VMEM((SCH, HH), jnp.float32),
        pltpu.VMEM((SCH, HH), jnp.float32),
        pltpu.VMEM_SHARED((N_NODE, HH), jnp.float32),
        pltpu.SemaphoreType.DMA,
        pltpu.SemaphoreType.DMA,
    ],
)(_segsum_body)


def _pre_body(efeat, dstidx, zeros, zeros_chunk, sd_out,
              dst_v, fbuf, staging, accum, gsem):
    c = lax.axis_index("c")
    s = lax.axis_index("s")
    pltpu.sync_copy(dstidx.at[c, s], dst_v)
    pltpu.sync_copy(zeros_chunk, staging)
    one_hot = jnp.where(lax.iota(jnp.int32, 16) == 0,
                        jnp.float32(1.0), jnp.float32(0.0))

    def set_ones(j, carry):
        staging[j, pl.ds(D_E, 16)] = one_hot
        return carry

    lax.fori_loop(0, PCH, set_ones, 0)
    _tiled_init(s, zeros, [accum])
    plsc.subcore_barrier()

    def body(g, carry):
        pltpu.async_copy(efeat.at[c, s, g], fbuf, gsem).wait()

        def place(j, carry2):
            staging[j, pl.ds(0, D_E)] = fbuf[pl.ds(j * D_E, D_E)]
            return carry2

        lax.fori_loop(0, PCH, place, 0)
        pltpu.sync_copy(staging, accum.at[dst_v.at[g]], add=True)
        return carry

    lax.fori_loop(0, PNCH, body, 0)
    plsc.subcore_barrier()
    _tiled_copy_out(s, [(accum, lambda o, n: sd_out.at[c, pl.ds(o, n)])])


_precompute = functools.partial(
    pl.kernel,
    out_type=jax.ShapeDtypeStruct((2, N_NODE, HH), jnp.float32),
    mesh=_MESH,
    scratch_types=[
        pltpu.VMEM((PNCH, PCH), jnp.int32),
        pltpu.VMEM((PCH * D_E,), jnp.float32),
        pltpu.VMEM((PCH, HH), jnp.float32),
        pltpu.VMEM_SHARED((N_NODE, HH), jnp.float32),
        pltpu.SemaphoreType.DMA,
    ],
)(_pre_body)


_BN = 2000


def _embed_body(h_ref, w_ref, b_ref, out_ref):
    x = jnp.dot(h_ref[...], w_ref[...],
                preferred_element_type=jnp.float32) + b_ref[...]
    out_ref[0] = x[:, :HH]
    out_ref[1] = x[:, HH:]


def _embed(h, W, b):
    n, d = h.shape
    return pl.pallas_call(
        _embed_body,
        grid=(n // _BN,),
        in_specs=[pl.BlockSpec((_BN, d), lambda i: (i, 0)),
                  pl.BlockSpec((d, H), lambda i: (0, 0)),
                  pl.BlockSpec((1, H), lambda i: (0, 0))],
        out_specs=pl.BlockSpec((2, _BN, HH), lambda i: (0, i, 0)),
        out_shape=jax.ShapeDtypeStruct((2, n, HH), jnp.float32),
    )(h, W, b.reshape(1, H))


def _conv_update_body(split_out, h2, s2, sd, wm, ws, wn, bm, bs, bn,
                      out_ref):
    Wm = wm[...]
    h = jnp.concatenate([h2[0], h2[1]], axis=1)
    Sg = jnp.concatenate([s2[0], s2[1]], axis=1)
    sdv = sd[...]
    sef = sdv[:, :D_E]
    degv = sdv[:, D_E:D_E + 1]
    agg_u = (jnp.dot(Sg, Wm[:H], preferred_element_type=jnp.float32)
             + jnp.dot(sef, Wm[H:], preferred_element_type=jnp.float32)
             + degv * bm[...])
    agg = agg_u / jnp.maximum(degv, 1.0)
    x = (jnp.dot(h, ws[...], preferred_element_type=jnp.float32)
         + jnp.dot(agg, wn[...], preferred_element_type=jnp.float32)
         + bs[...] + bn[...])
    x = jnp.maximum(x, 0.0)
    nrm = jnp.sqrt(jnp.sum(x * x, axis=1, keepdims=True))
    x = x / (nrm + 1e-6)
    if split_out:
        out_ref[0] = x[:, :HH]
        out_ref[1] = x[:, HH:]
    else:
        out_ref[...] = x


def _conv_update(h2, s2, sd, Wm, bm, Ws, bs, Wn, bn, split_out):
    if split_out:
        out_spec = pl.BlockSpec((2, _BN, HH), lambda i: (0, i, 0))
        out_shape = jax.ShapeDtypeStruct((2, N_NODE, HH), jnp.float32)
    else:
        out_spec = pl.BlockSpec((_BN, H), lambda i: (i, 0))
        out_shape = jax.ShapeDtypeStruct((N_NODE, H), jnp.float32)
    return pl.pallas_call(
        functools.partial(_conv_update_body, split_out),
        grid=(N_NODE // _BN,),
        in_specs=[pl.BlockSpec((2, _BN, HH), lambda i: (0, i, 0)),
                  pl.BlockSpec((2, _BN, HH), lambda i: (0, i, 0)),
                  pl.BlockSpec((_BN, HH), lambda i: (i, 0)),
                  pl.BlockSpec((H + D_E, H), lambda i: (0, 0)),
                  pl.BlockSpec((H, H), lambda i: (0, 0)),
                  pl.BlockSpec((H, H), lambda i: (0, 0)),
                  pl.BlockSpec((1, H), lambda i: (0, 0)),
                  pl.BlockSpec((1, H), lambda i: (0, 0)),
                  pl.BlockSpec((1, H), lambda i: (0, 0))],
        out_specs=out_spec,
        out_shape=out_shape,
    )(h2, s2, sd, Wm, Ws, Wn,
      bm.reshape(1, H), bs.reshape(1, H), bn.reshape(1, H))


def kernel(h_customer, h_product, edge_index_c2p, edge_index_p2c,
           edge_feat_c2p, edge_feat_p2c,
           W_user, b_user, W_item, b_item,
           W_msg_c2p, b_msg_c2p, W_self_c2p, b_self_c2p, W_neigh_c2p, b_neigh_c2p,
           W_msg_p2c, b_msg_p2c, W_self_p2c, b_self_p2c, W_neigh_p2c, b_neigh_p2c):
    f32 = jnp.float32
    i32 = jnp.int32
    N = N_NODE

    src_cp = edge_index_c2p[0].astype(i32)
    dst_cp = edge_index_c2p[1].astype(i32)
    src_pc = edge_index_p2c[0].astype(i32)
    dst_pc = edge_index_p2c[1].astype(i32)

    srcidx_cp = jnp.stack([src_cp, src_cp + N]).reshape(2, NTILE, EPT)
    srcidx_pc = jnp.stack([src_pc, src_pc + N]).reshape(2, NTILE, EPT)
    dstidx_cp = dst_cp.reshape(NTILE, SNCH, SCH)
    dstidx_pc = dst_pc.reshape(NTILE, SNCH, SCH)

    efeat_both = jnp.stack([edge_feat_c2p.reshape(NTILE, PNCH, PCH * D_E),
                            edge_feat_p2c.reshape(NTILE, PNCH, PCH * D_E)])
    pre_dst_both = jnp.stack([dst_cp.reshape(NTILE, PNCH, PCH),
                              dst_pc.reshape(NTILE, PNCH, PCH)])
    zeros_hh = jnp.zeros((ALN, HH), f32)

    sefdeg = _precompute(efeat_both, pre_dst_both, zeros_hh,
                         jnp.zeros((PCH, HH), f32))
    sd_p = sefdeg[0]
    sd_c = sefdeg[1]

    hc2 = _embed(h_customer, W_user, b_user)
    hp2 = _embed(h_product, W_item, b_item)

    prev_sc = sefdeg
    for l in range(N_LAYERS_K):
        split = l < N_LAYERS_K - 1
        hc2_d, _ = lax.optimization_barrier((hc2, prev_sc))
        s_cp = _segsum(hc2_d.reshape(2 * N, HH), srcidx_cp, dstidx_cp,
                       zeros_hh)
        hp2_d, _ = lax.optimization_barrier((hp2, s_cp))
        s_pc = _segsum(hp2_d.reshape(2 * N, HH), srcidx_pc, dstidx_pc,
                       zeros_hh)
        prev_sc = s_pc
        new_hp = _conv_update(hp2, s_cp, sd_p,
                              W_msg_c2p[l], b_msg_c2p[l], W_self_c2p[l],
                              b_self_c2p[l], W_neigh_c2p[l], b_neigh_c2p[l],
                              split)
        new_hc = _conv_update(hc2, s_pc, sd_c,
                              W_msg_p2c[l], b_msg_p2c[l], W_self_p2c[l],
                              b_self_p2c[l], W_neigh_p2c[l], b_neigh_p2c[l],
                              split)
        hc2, hp2 = new_hc, new_hp

    return hc2, hp2


# speedup vs baseline: 1.3800x; 1.1732x over previous
import functools

import jax
import jax.numpy as jnp
from jax import lax
from jax.experimental import pallas as pl
from jax.experimental.pallas import tpu as pltpu
from jax.experimental.pallas import tpu_sc as plsc

N_NODE = 10000
E_EDGES = 160000
H = 256
HH = 128
D_E = 16
N_LAYERS_K = 2
NTILE = 16
EPT = E_EDGES // NTILE

SCH = 80
SNCH = EPT // SCH

PCH = 80
PNCH = EPT // PCH

ALN = 624
TAIL = N_NODE - ALN * NTILE

_MESH = plsc.VectorSubcoreMesh(core_axis_name="c", subcore_axis_name="s")


def _tiled_init(s, zeros, accs):
    for acc in accs:
        pltpu.sync_copy(zeros.at[pl.ds(0, ALN)], acc.at[pl.ds(s * ALN, ALN)])

    @pl.when(s == NTILE - 1)
    def _():
        for acc in accs:
            pltpu.sync_copy(zeros.at[pl.ds(0, TAIL)],
                            acc.at[pl.ds(ALN * NTILE, TAIL)])


def _tiled_copy_out(s, outs):
    for acc, out_slicer in outs:
        pltpu.sync_copy(acc.at[pl.ds(s * ALN, ALN)], out_slicer(s * ALN, ALN))

    @pl.when(s == NTILE - 1)
    def _():
        for acc, out_slicer in outs:
            pltpu.sync_copy(acc.at[pl.ds(ALN * NTILE, TAIL)],
                            out_slicer(ALN * NTILE, TAIL))


def _segsum_body(htable, srcidx, dstidx, zeros, out,
                 src_v, dst_v, buf_a, buf_b, accum, gsem_a, gsem_b):
    c = lax.axis_index("c")
    s = lax.axis_index("s")
    pltpu.sync_copy(srcidx.at[c, s], src_v)
    pltpu.sync_copy(dstidx.at[s], dst_v)
    _tiled_init(s, zeros, [accum])
    plsc.subcore_barrier()

    def gather(g, buf, sem):
        pltpu.async_copy(htable.at[src_v.at[pl.ds(g * SCH, SCH)]], buf, sem)

    def gather_wait(g, buf, sem):
        pltpu.make_async_copy(htable.at[src_v.at[pl.ds(g * SCH, SCH)]],
                              buf, sem).wait()

    gather(0, buf_a, gsem_a)
    gather(1, buf_b, gsem_b)

    def body(t, carry):
        g = t * 2
        gather_wait(g, buf_a, gsem_a)
        pltpu.sync_copy(buf_a, accum.at[dst_v.at[g]], add=True)

        @pl.when(g + 2 < SNCH)
        def _():
            gather(g + 2, buf_a, gsem_a)

        gather_wait(g + 1, buf_b, gsem_b)
        pltpu.sync_copy(buf_b, accum.at[dst_v.at[g + 1]], add=True)

        @pl.when(g + 3 < SNCH)
        def _():
            gather(g + 3, buf_b, gsem_b)

        return carry

    lax.fori_loop(0, SNCH // 2, body, 0)

    gather_wait(SNCH - 1, buf_a, gsem_a)
    pltpu.sync_copy(buf_a, accum.at[dst_v.at[SNCH - 1]], add=True)

    plsc.subcore_barrier()
    _tiled_copy_out(s, [(accum, lambda o, n: out.at[c, pl.ds(o, n)])])


_segsum = functools.partial(
    pl.kernel,
    out_type=jax.ShapeDtypeStruct((2, N_NODE, HH), jnp.float32),
    mesh=_MESH,
    scratch_types=[
        pltpu.VMEM((EPT,), jnp.int32),
        pltpu.VMEM((SNCH, SCH), jnp.int32),
        pltpu.VMEM((SCH, HH), jnp.float32),
        pltpu.VMEM((SCH, HH), jnp.float32),
        pltpu.VMEM_SHARED((N_NODE, HH), jnp.float32),
        pltpu.SemaphoreType.DMA,
        pltpu.SemaphoreType.DMA,
    ],
)(_segsum_body)


def _pre_body(efeat, dstidx, zeros, zeros_chunk, sd_out,
              dst_v, fbuf, staging, accum, gsem):
    c = lax.axis_index("c")
    s = lax.axis_index("s")
    pltpu.sync_copy(dstidx.at[c, s], dst_v)
    pltpu.sync_copy(zeros_chunk, staging)
    one_hot = jnp.where(lax.iota(jnp.int32, 16) == 0,
                        jnp.float32(1.0), jnp.float32(0.0))

    def set_ones(j, carry):
        staging[j, pl.ds(D_E, 16)] = one_hot
        return carry

    lax.fori_loop(0, PCH, set_ones, 0)
    _tiled_init(s, zeros, [accum])
    plsc.subcore_barrier()

    def body(g, carry):
        pltpu.async_copy(efeat.at[c, s, g], fbuf, gsem).wait()

        def place(j, carry2):
            staging[j, pl.ds(0, D_E)] = fbuf[pl.ds(j * D_E, D_E)]
            return carry2

        lax.fori_loop(0, PCH, place, 0)
        pltpu.sync_copy(staging, accum.at[dst_v.at[g]], add=True)
        return carry

    lax.fori_loop(0, PNCH, body, 0)
    plsc.subcore_barrier()
    _tiled_copy_out(s, [(accum, lambda o, n: sd_out.at[c, pl.ds(o, n)])])


_precompute = functools.partial(
    pl.kernel,
    out_type=jax.ShapeDtypeStruct((2, N_NODE, HH), jnp.float32),
    mesh=_MESH,
    scratch_types=[
        pltpu.VMEM((PNCH, PCH), jnp.int32),
        pltpu.VMEM((PCH * D_E,), jnp.float32),
        pltpu.VMEM((PCH, HH), jnp.float32),
        pltpu.VMEM_SHARED((N_NODE, HH), jnp.float32),
        pltpu.SemaphoreType.DMA,
    ],
)(_pre_body)


_BN = 2000


def _embed_body(h_ref, w_ref, b_ref, out_ref):
    x = jnp.dot(h_ref[...], w_ref[...],
                preferred_element_type=jnp.float32) + b_ref[...]
    out_ref[0] = x[:, :HH]
    out_ref[1] = x[:, HH:]


def _embed(h, W, b):
    n, d = h.shape
    return pl.pallas_call(
        _embed_body,
        grid=(n // _BN,),
        in_specs=[pl.BlockSpec((_BN, d), lambda i: (i, 0)),
                  pl.BlockSpec((d, H), lambda i: (0, 0)),
                  pl.BlockSpec((1, H), lambda i: (0, 0))],
        out_specs=pl.BlockSpec((2, _BN, HH), lambda i: (0, i, 0)),
        out_shape=jax.ShapeDtypeStruct((2, n, HH), jnp.float32),
    )(h, W, b.reshape(1, H))


def _conv_update_body(split_out, h2, s2, sd, wm, ws, wn, bm, bs, bn,
                      out_ref):
    Wm = wm[...]
    h = jnp.concatenate([h2[0], h2[1]], axis=1)
    Sg = jnp.concatenate([s2[0], s2[1]], axis=1)
    sdv = sd[...]
    sef = sdv[:, :D_E]
    degv = sdv[:, D_E:D_E + 1]
    agg_u = (jnp.dot(Sg, Wm[:H], preferred_element_type=jnp.float32)
             + jnp.dot(sef, Wm[H:], preferred_element_type=jnp.float32)
             + degv * bm[...])
    agg = agg_u / jnp.maximum(degv, 1.0)
    x = (jnp.dot(h, ws[...], preferred_element_type=jnp.float32)
         + jnp.dot(agg, wn[...], preferred_element_type=jnp.float32)
         + bs[...] + bn[...])
    x = jnp.maximum(x, 0.0)
    nrm = jnp.sqrt(jnp.sum(x * x, axis=1, keepdims=True))
    x = x / (nrm + 1e-6)
    if split_out:
        out_ref[0] = x[:, :HH]
        out_ref[1] = x[:, HH:]
    else:
        out_ref[...] = x


def _conv_update(h2, s2, sd, Wm, bm, Ws, bs, Wn, bn, split_out):
    if split_out:
        out_spec = pl.BlockSpec((2, _BN, HH), lambda i: (0, i, 0))
        out_shape = jax.ShapeDtypeStruct((2, N_NODE, HH), jnp.float32)
    else:
        out_spec = pl.BlockSpec((_BN, H), lambda i: (i, 0))
        out_shape = jax.ShapeDtypeStruct((N_NODE, H), jnp.float32)
    return pl.pallas_call(
        functools.partial(_conv_update_body, split_out),
        grid=(N_NODE // _BN,),
        in_specs=[pl.BlockSpec((2, _BN, HH), lambda i: (0, i, 0)),
                  pl.BlockSpec((2, _BN, HH), lambda i: (0, i, 0)),
                  pl.BlockSpec((_BN, HH), lambda i: (i, 0)),
                  pl.BlockSpec((H + D_E, H), lambda i: (0, 0)),
                  pl.BlockSpec((H, H), lambda i: (0, 0)),
                  pl.BlockSpec((H, H), lambda i: (0, 0)),
                  pl.BlockSpec((1, H), lambda i: (0, 0)),
                  pl.BlockSpec((1, H), lambda i: (0, 0)),
                  pl.BlockSpec((1, H), lambda i: (0, 0))],
        out_specs=out_spec,
        out_shape=out_shape,
    )(h2, s2, sd, Wm, Ws, Wn,
      bm.reshape(1, H), bs.reshape(1, H), bn.reshape(1, H))


def kernel(h_customer, h_product, edge_index_c2p, edge_index_p2c,
           edge_feat_c2p, edge_feat_p2c,
           W_user, b_user, W_item, b_item,
           W_msg_c2p, b_msg_c2p, W_self_c2p, b_self_c2p, W_neigh_c2p, b_neigh_c2p,
           W_msg_p2c, b_msg_p2c, W_self_p2c, b_self_p2c, W_neigh_p2c, b_neigh_p2c):
    f32 = jnp.float32
    i32 = jnp.int32
    N = N_NODE

    src_cp = edge_index_c2p[0].astype(i32)
    dst_cp = edge_index_c2p[1].astype(i32)
    src_pc = edge_index_p2c[0].astype(i32)
    dst_pc = edge_index_p2c[1].astype(i32)

    srcidx_cp = jnp.stack([src_cp, src_cp + N]).reshape(2, NTILE, EPT)
    srcidx_pc = jnp.stack([src_pc, src_pc + N]).reshape(2, NTILE, EPT)
    dstidx_cp = dst_cp.reshape(NTILE, SNCH, SCH)
    dstidx_pc = dst_pc.reshape(NTILE, SNCH, SCH)

    efeat_both = jnp.stack([edge_feat_c2p.reshape(NTILE, PNCH, PCH * D_E),
                            edge_feat_p2c.reshape(NTILE, PNCH, PCH * D_E)])
    pre_dst_both = jnp.stack([dst_cp.reshape(NTILE, PNCH, PCH),
                              dst_pc.reshape(NTILE, PNCH, PCH)])
    zeros_hh = jnp.zeros((ALN, HH), f32)

    sefdeg = _precompute(efeat_both, pre_dst_both, zeros_hh,
                         jnp.zeros((PCH, HH), f32))
    sd_p = sefdeg[0]
    sd_c = sefdeg[1]

    hc2 = _embed(h_customer, W_user, b_user)
    hp2 = _embed(h_product, W_item, b_item)

    prev_sc = sefdeg
    for l in range(N_LAYERS_K):
        split = l < N_LAYERS_K - 1
        hc2_d, _ = lax.optimization_barrier((hc2, prev_sc))
        s_cp = _segsum(hc2_d.reshape(2 * N, HH), srcidx_cp, dstidx_cp,
                       zeros_hh)
        hp2_d, _ = lax.optimization_barrier((hp2, s_cp))
        s_pc = _segsum(hp2_d.reshape(2 * N, HH), srcidx_pc, dstidx_pc,
                       zeros_hh)
        prev_sc = s_pc
        new_hp = _conv_update(hp2, s_cp, sd_p,
                              W_msg_c2p[l], b_msg_c2p[l], W_self_c2p[l],
                              b_self_c2p[l], W_neigh_c2p[l], b_neigh_c2p[l],
                              split)
        new_hc = _conv_update(hc2, s_pc, sd_c,
                              W_msg_p2c[l], b_msg_p2c[l], W_self_p2c[l],
                              b_self_p2c[l], W_neigh_p2c[l], b_neigh_p2c[l],
                              split)
        hc2, hp2 = new_hc, new_hp

    return hc2, hp2


# speedup vs baseline: 1.6598x; 1.2027x over previous
import functools

import jax
import jax.numpy as jnp
from jax import lax
from jax.experimental import pallas as pl
from jax.experimental.pallas import tpu as pltpu
from jax.experimental.pallas import tpu_sc as plsc

N_NODE = 10000
E_EDGES = 160000
H = 256
HH = 128
D_E = 16
N_LAYERS_K = 2
NTILE = 16
EPT = E_EDGES // NTILE

SCH = 80
SNCH = EPT // SCH

PCH = 80
PNCH = EPT // PCH

ALN = 624
TAIL = N_NODE - ALN * NTILE

_MESH = plsc.VectorSubcoreMesh(core_axis_name="c", subcore_axis_name="s")


def _tiled_init(s, zeros, accs):
    for acc in accs:
        pltpu.sync_copy(zeros.at[pl.ds(0, ALN)], acc.at[pl.ds(s * ALN, ALN)])

    @pl.when(s == NTILE - 1)
    def _():
        for acc in accs:
            pltpu.sync_copy(zeros.at[pl.ds(0, TAIL)],
                            acc.at[pl.ds(ALN * NTILE, TAIL)])


def _tiled_copy_out(s, outs):
    for acc, out_slicer in outs:
        pltpu.sync_copy(acc.at[pl.ds(s * ALN, ALN)], out_slicer(s * ALN, ALN))

    @pl.when(s == NTILE - 1)
    def _():
        for acc, out_slicer in outs:
            pltpu.sync_copy(acc.at[pl.ds(ALN * NTILE, TAIL)],
                            out_slicer(ALN * NTILE, TAIL))


def _segsum_body(htable, srcidx, dstidx, zeros, out,
                 src_v, dst_v, buf_a, buf_b, accum, gsem_a, gsem_b):
    c = lax.axis_index("c")
    s = lax.axis_index("s")
    pltpu.sync_copy(srcidx.at[c, s], src_v)
    pltpu.sync_copy(dstidx.at[s], dst_v)
    _tiled_init(s, zeros, [accum])
    plsc.subcore_barrier()

    def gather(g, buf, sem):
        pltpu.async_copy(htable.at[src_v.at[pl.ds(g * SCH, SCH)]], buf, sem)

    def gather_wait(g, buf, sem):
        pltpu.make_async_copy(htable.at[src_v.at[pl.ds(g * SCH, SCH)]],
                              buf, sem).wait()

    gather(0, buf_a, gsem_a)
    gather(1, buf_b, gsem_b)

    def body(t, carry):
        g = t * 2
        gather_wait(g, buf_a, gsem_a)
        pltpu.sync_copy(buf_a, accum.at[dst_v.at[g]], add=True)

        @pl.when(g + 2 < SNCH)
        def _():
            gather(g + 2, buf_a, gsem_a)

        gather_wait(g + 1, buf_b, gsem_b)
        pltpu.sync_copy(buf_b, accum.at[dst_v.at[g + 1]], add=True)

        @pl.when(g + 3 < SNCH)
        def _():
            gather(g + 3, buf_b, gsem_b)

        return carry

    lax.fori_loop(0, SNCH // 2, body, 0)

    gather_wait(SNCH - 1, buf_a, gsem_a)
    pltpu.sync_copy(buf_a, accum.at[dst_v.at[SNCH - 1]], add=True)

    plsc.subcore_barrier()
    _tiled_copy_out(s, [(accum, lambda o, n: out.at[c, pl.ds(o, n)])])


_segsum = functools.partial(
    pl.kernel,
    out_type=jax.ShapeDtypeStruct((2, N_NODE, HH), jnp.float32),
    mesh=_MESH,
    scratch_types=[
        pltpu.VMEM((EPT,), jnp.int32),
        pltpu.VMEM((SNCH, SCH), jnp.int32),
        pltpu.VMEM((SCH, HH), jnp.float32),
        pltpu.VMEM((SCH, HH), jnp.float32),
        pltpu.VMEM_SHARED((N_NODE, HH), jnp.float32),
        pltpu.SemaphoreType.DMA,
        pltpu.SemaphoreType.DMA,
    ],
)(_segsum_body)


def _pre_body(efeat, dstidx, zeros, zeros_chunk, sd_out,
              dst_v, fbufs, stagings, accum, efsems, ssems):
    c = lax.axis_index("c")
    s = lax.axis_index("s")
    pltpu.sync_copy(dstidx.at[c, s], dst_v)
    one_hot = jnp.where(lax.iota(jnp.int32, 16) == 0,
                        jnp.float32(1.0), jnp.float32(0.0))
    for staging in stagings:
        pltpu.sync_copy(zeros_chunk, staging)

        def set_ones(j, carry):
            staging[j, pl.ds(D_E, 16)] = one_hot
            return carry

        lax.fori_loop(0, PCH, set_ones, 0)
    _tiled_init(s, zeros, [accum])

    def ef_fetch(g, b):
        pltpu.async_copy(efeat.at[c, s, g], fbufs[b], efsems[b])

    def ef_wait(g, b):
        pltpu.make_async_copy(efeat.at[c, s, g], fbufs[b], efsems[b]).wait()

    def place(b):
        def step(j, carry):
            stagings[b][j, pl.ds(0, D_E)] = fbufs[b][0, pl.ds(j * D_E, D_E)]
            return carry

        lax.fori_loop(0, PCH, step, 0)

    def scatter(g, b):
        pltpu.async_copy(stagings[b], accum.at[dst_v.at[g, 0]], ssems[b],
                         add=True)

    def scatter_wait(g, b):
        pltpu.make_async_copy(stagings[b], accum.at[dst_v.at[g, 0]],
                              ssems[b]).wait()

    ef_fetch(0, 0)
    ef_fetch(1, 1)
    plsc.subcore_barrier()

    def body(t, carry):
        g = t * 2
        for b in range(2):
            @pl.when(t > 0)
            def _():
                scatter_wait(g + b - 2, b)

            ef_wait(g + b, b)
            place(b)

            @pl.when(g + b + 2 < PNCH)
            def _():
                ef_fetch(g + b + 2, b)

            scatter(g + b, b)
        return carry

    lax.fori_loop(0, PNCH // 2, body, 0)

    scatter_wait(PNCH - 3, 0)
    ef_wait(PNCH - 1, 0)
    place(0)
    scatter(PNCH - 1, 0)
    scatter_wait(PNCH - 1, 0)
    scatter_wait(PNCH - 2, 1)

    plsc.subcore_barrier()
    _tiled_copy_out(s, [(accum, lambda o, n: sd_out.at[c, pl.ds(o, n)])])


_precompute = functools.partial(
    pl.kernel,
    out_type=jax.ShapeDtypeStruct((2, N_NODE, HH), jnp.float32),
    mesh=_MESH,
    scratch_types=[
        pltpu.VMEM((PNCH, 1, PCH), jnp.int32),
        [pltpu.VMEM((1, PCH * D_E), jnp.float32)] * 2,
        [pltpu.VMEM((PCH, HH), jnp.float32)] * 2,
        pltpu.VMEM_SHARED((N_NODE, HH), jnp.float32),
        [pltpu.SemaphoreType.DMA] * 2,
        [pltpu.SemaphoreType.DMA] * 2,
    ],
)(_pre_body)


_BN = 2000


def _embed_body(h_ref, w_ref, b_ref, out_ref):
    x = jnp.dot(h_ref[...], w_ref[...],
                preferred_element_type=jnp.float32) + b_ref[...]
    out_ref[0] = x[:, :HH]
    out_ref[1] = x[:, HH:]


def _embed(h, W, b):
    n, d = h.shape
    return pl.pallas_call(
        _embed_body,
        grid=(n // _BN,),
        in_specs=[pl.BlockSpec((_BN, d), lambda i: (i, 0)),
                  pl.BlockSpec((d, H), lambda i: (0, 0)),
                  pl.BlockSpec((1, H), lambda i: (0, 0))],
        out_specs=pl.BlockSpec((2, _BN, HH), lambda i: (0, i, 0)),
        out_shape=jax.ShapeDtypeStruct((2, n, HH), jnp.float32),
    )(h, W, b.reshape(1, H))


def _conv_update_body(split_out, h2, s2, sd, wm, ws, wn, bm, bs, bn,
                      out_ref):
    Wm = wm[...]
    h = jnp.concatenate([h2[0], h2[1]], axis=1)
    Sg = jnp.concatenate([s2[0], s2[1]], axis=1)
    sdv = sd[...]
    sef = sdv[:, :D_E]
    degv = sdv[:, D_E:D_E + 1]
    agg_u = (jnp.dot(Sg, Wm[:H], preferred_element_type=jnp.float32)
             + jnp.dot(sef, Wm[H:], preferred_element_type=jnp.float32)
             + degv * bm[...])
    agg = agg_u / jnp.maximum(degv, 1.0)
    x = (jnp.dot(h, ws[...], preferred_element_type=jnp.float32)
         + jnp.dot(agg, wn[...], preferred_element_type=jnp.float32)
         + bs[...] + bn[...])
    x = jnp.maximum(x, 0.0)
    nrm = jnp.sqrt(jnp.sum(x * x, axis=1, keepdims=True))
    x = x / (nrm + 1e-6)
    if split_out:
        out_ref[0] = x[:, :HH]
        out_ref[1] = x[:, HH:]
    else:
        out_ref[...] = x


def _conv_update(h2, s2, sd, Wm, bm, Ws, bs, Wn, bn, split_out):
    if split_out:
        out_spec = pl.BlockSpec((2, _BN, HH), lambda i: (0, i, 0))
        out_shape = jax.ShapeDtypeStruct((2, N_NODE, HH), jnp.float32)
    else:
        out_spec = pl.BlockSpec((_BN, H), lambda i: (i, 0))
        out_shape = jax.ShapeDtypeStruct((N_NODE, H), jnp.float32)
    return pl.pallas_call(
        functools.partial(_conv_update_body, split_out),
        grid=(N_NODE // _BN,),
        in_specs=[pl.BlockSpec((2, _BN, HH), lambda i: (0, i, 0)),
                  pl.BlockSpec((2, _BN, HH), lambda i: (0, i, 0)),
                  pl.BlockSpec((_BN, HH), lambda i: (i, 0)),
                  pl.BlockSpec((H + D_E, H), lambda i: (0, 0)),
                  pl.BlockSpec((H, H), lambda i: (0, 0)),
                  pl.BlockSpec((H, H), lambda i: (0, 0)),
                  pl.BlockSpec((1, H), lambda i: (0, 0)),
                  pl.BlockSpec((1, H), lambda i: (0, 0)),
                  pl.BlockSpec((1, H), lambda i: (0, 0))],
        out_specs=out_spec,
        out_shape=out_shape,
    )(h2, s2, sd, Wm, Ws, Wn,
      bm.reshape(1, H), bs.reshape(1, H), bn.reshape(1, H))


def kernel(h_customer, h_product, edge_index_c2p, edge_index_p2c,
           edge_feat_c2p, edge_feat_p2c,
           W_user, b_user, W_item, b_item,
           W_msg_c2p, b_msg_c2p, W_self_c2p, b_self_c2p, W_neigh_c2p, b_neigh_c2p,
           W_msg_p2c, b_msg_p2c, W_self_p2c, b_self_p2c, W_neigh_p2c, b_neigh_p2c):
    f32 = jnp.float32
    i32 = jnp.int32
    N = N_NODE

    src_cp = edge_index_c2p[0].astype(i32)
    dst_cp = edge_index_c2p[1].astype(i32)
    src_pc = edge_index_p2c[0].astype(i32)
    dst_pc = edge_index_p2c[1].astype(i32)

    srcidx_cp = jnp.stack([src_cp, src_cp + N]).reshape(2, NTILE, EPT)
    srcidx_pc = jnp.stack([src_pc, src_pc + N]).reshape(2, NTILE, EPT)
    dstidx_cp = dst_cp.reshape(NTILE, SNCH, SCH)
    dstidx_pc = dst_pc.reshape(NTILE, SNCH, SCH)

    efeat_both = jnp.stack(
        [edge_feat_c2p.reshape(NTILE, PNCH, 1, PCH * D_E),
         edge_feat_p2c.reshape(NTILE, PNCH, 1, PCH * D_E)])
    pre_dst_both = jnp.stack([dst_cp.reshape(NTILE, PNCH, 1, PCH),
                              dst_pc.reshape(NTILE, PNCH, 1, PCH)])
    zeros_hh = jnp.zeros((ALN, HH), f32)

    sefdeg = _precompute(efeat_both, pre_dst_both, zeros_hh,
                         jnp.zeros((PCH, HH), f32))
    sd_p = sefdeg[0]
    sd_c = sefdeg[1]

    hc2 = _embed(h_customer, W_user, b_user)
    hp2 = _embed(h_product, W_item, b_item)

    prev_sc = sefdeg
    for l in range(N_LAYERS_K):
        split = l < N_LAYERS_K - 1
        hc2_d, _ = lax.optimization_barrier((hc2, prev_sc))
        s_cp = _segsum(hc2_d.reshape(2 * N, HH), srcidx_cp, dstidx_cp,
                       zeros_hh)
        hp2_d, _ = lax.optimization_barrier((hp2, s_cp))
        s_pc = _segsum(hp2_d.reshape(2 * N, HH), srcidx_pc, dstidx_pc,
                       zeros_hh)
        prev_sc = s_pc
        new_hp = _conv_update(hp2, s_cp, sd_p,
                              W_msg_c2p[l], b_msg_c2p[l], W_self_c2p[l],
                              b_self_c2p[l], W_neigh_c2p[l], b_neigh_c2p[l],
                              split)
        new_hc = _conv_update(hc2, s_pc, sd_c,
                              W_msg_p2c[l], b_msg_p2c[l], W_self_p2c[l],
                              b_self_p2c[l], W_neigh_p2c[l], b_neigh_p2c[l],
                              split)
        hc2, hp2 = new_hc, new_hp

    return hc2, hp2
